# Initial kernel scaffold; baseline (speedup 1.0000x reference)
import jax
import jax.numpy as jnp
from jax.experimental import pallas as pl


def kernel(x, edge_index, batch, W1, b1, W2, b2, W3, b3, Wg, bg, Wc1, bc1, Wc2, bc2):
    raise NotImplementedError("write your pallas kernel here")


# speedup vs baseline: 7.0474x; 7.0474x over previous
import functools

import jax
import jax.numpy as jnp
from jax import lax
from jax.experimental import pallas as pl
from jax.experimental.pallas import tpu as pltpu
from jax.experimental.pallas import tpu_sc as plsc

N = 10000
E = 320000
G = 64
IN = 128
HID = 200
DH = 128
D = 2 * DH
NPAD = 10080
RB = 336
GRID = NPAD // RB

NC, NS = 2, 16
NW = NC * NS
K = 128
E_PAD = ((E + NW * K - 1) // (NW * K)) * (NW * K)
EPS = E_PAD // NS
NCHUNK_P = EPS // K
EPW = E_PAD // NW
NCHUNK_D = EPW // K
WROWS = 632
ZR = 8

_mesh = plsc.VectorSubcoreMesh(core_axis_name="c", subcore_axis_name="s")


@functools.partial(
    pl.kernel,
    out_type=jax.ShapeDtypeStruct((NC, NPAD, 16), jnp.float32),
    mesh=_mesh,
    scratch_types=[
        pltpu.VMEM((K,), jnp.int32),
        pltpu.VMEM((K, 16), jnp.float32),
        pltpu.VMEM((ZR, 16), jnp.float32),
        pltpu.VMEM_SHARED((NPAD, 16), jnp.float32),
        pltpu.SemaphoreType.DMA,
    ],
)
def _degree(dst_hbm, out_hbm, idx_d, ones_b, zbuf, acc, sem):
    c = lax.axis_index("c")
    s = lax.axis_index("s")
    one16 = jnp.ones((16,), jnp.float32)
    zero16 = jnp.zeros((16,), jnp.float32)
    for i in range(K):
        ones_b[i, :] = one16
    for i in range(ZR):
        zbuf[i, :] = zero16
    base_row = jnp.minimum(s * WROWS, NPAD - WROWS)

    def zinit(j, carry):
        pltpu.sync_copy(zbuf, acc.at[pl.ds(base_row + j * ZR, ZR)])
        return carry

    lax.fori_loop(0, WROWS // ZR, zinit, 0)
    plsc.subcore_barrier()

    ebase = (c * NS + s) * EPW

    def body(g, carry):
        pltpu.sync_copy(dst_hbm.at[pl.ds(ebase + g * K, K)], idx_d)
        pltpu.sync_copy(ones_b, acc.at[idx_d], add=True)
        return carry

    lax.fori_loop(0, NCHUNK_D, body, 0)
    plsc.subcore_barrier()
    pltpu.sync_copy(acc.at[pl.ds(base_row, WROWS)],
                    out_hbm.at[c, pl.ds(base_row, WROWS)])


@functools.partial(
    pl.kernel,
    out_type=jax.ShapeDtypeStruct((NC, NPAD, DH), jnp.float32),
    mesh=_mesh,
    scratch_types=[
        pltpu.VMEM((K,), jnp.int32),
        pltpu.VMEM((K,), jnp.int32),
        pltpu.VMEM((K, DH), jnp.float32),
        pltpu.VMEM((ZR, DH), jnp.float32),
        pltpu.VMEM_SHARED((NPAD, DH), jnp.float32),
        pltpu.SemaphoreType.DMA,
    ],
)
def _propagate(src_hbm, dst_hbm, p_hbm, out_hbm, idx_s, idx_d, rows, zbuf,
               acc, sem):
    c = lax.axis_index("c")
    s = lax.axis_index("s")
    zero16 = jnp.zeros((16,), jnp.float32)
    for i in range(ZR):
        for k in range(DH // 16):
            zbuf[i, pl.ds(k * 16, 16)] = zero16
    base_row = jnp.minimum(s * WROWS, NPAD - WROWS)

    def zinit(j, carry):
        pltpu.sync_copy(zbuf, acc.at[pl.ds(base_row + j * ZR, ZR)])
        return carry

    lax.fori_loop(0, WROWS // ZR, zinit, 0)
    plsc.subcore_barrier()

    ebase = s * EPS

    def body(g, carry):
        off = ebase + g * K
        pltpu.sync_copy(src_hbm.at[pl.ds(off, K)], idx_s)
        pltpu.sync_copy(dst_hbm.at[pl.ds(off, K)], idx_d)
        pltpu.async_copy(p_hbm.at[c].at[idx_s], rows, sem).wait()
        pltpu.sync_copy(rows, acc.at[idx_d], add=True)
        return carry

    lax.fori_loop(0, NCHUNK_P, body, 0)
    plsc.subcore_barrier()
    pltpu.sync_copy(acc.at[pl.ds(base_row, WROWS)],
                    out_hbm.at[c, pl.ds(base_row, WROWS)])


def _t1_body(dp_ref, x_ref, w_ref, p_ref, dinv_ref):
    pid = pl.program_id(0)
    dp0 = dp_ref[0, :, :]
    dp1 = dp_ref[1, :, :]
    deg = 1.0 + dp0[:, :1] + dp1[:, :1]
    row = pid * RB + lax.broadcasted_iota(jnp.int32, (RB, 1), 0)
    dinv = jnp.where(row < N, lax.rsqrt(deg), 0.0)
    z = jnp.dot(x_ref[...], w_ref[...],
                preferred_element_type=jnp.float32) * dinv
    p_ref[0, :, :] = z[:, :DH]
    p_ref[1, :, :] = z[:, DH:]
    dinv_ref[...] = jnp.broadcast_to(dinv, (RB, 8))


def _t1(degp, xp, w1p):
    return pl.pallas_call(
        _t1_body,
        grid=(GRID,),
        in_specs=[
            pl.BlockSpec((NC, RB, 16), lambda i: (0, i, 0)),
            pl.BlockSpec((RB, IN), lambda i: (i, 0)),
            pl.BlockSpec((IN, D), lambda i: (0, 0)),
        ],
        out_specs=[
            pl.BlockSpec((NC, RB, DH), lambda i: (0, i, 0)),
            pl.BlockSpec((RB, 8), lambda i: (i, 0)),
        ],
        out_shape=[
            jax.ShapeDtypeStruct((NC, NPAD, DH), jnp.float32),
            jax.ShapeDtypeStruct((NPAD, 8), jnp.float32),
        ],
    )(degp, xp, w1p)


def _t2_body(p_ref, s_ref, dinv_ref, b_ref, w_ref, out_ref):
    dinv = dinv_ref[:, :1]
    ps = jnp.concatenate(
        [p_ref[0, :, :] + s_ref[0, :, :], p_ref[1, :, :] + s_ref[1, :, :]],
        axis=1)
    h = jnp.maximum(dinv * ps + b_ref[...], 0.0)
    z = jnp.dot(h, w_ref[...], preferred_element_type=jnp.float32) * dinv
    out_ref[0, :, :] = z[:, :DH]
    out_ref[1, :, :] = z[:, DH:]


def _t2(p, spart, dinv8, bprev, wnext):
    return pl.pallas_call(
        _t2_body,
        grid=(GRID,),
        in_specs=[
            pl.BlockSpec((NC, RB, DH), lambda i: (0, i, 0)),
            pl.BlockSpec((NC, RB, DH), lambda i: (0, i, 0)),
            pl.BlockSpec((RB, 8), lambda i: (i, 0)),
            pl.BlockSpec((1, D), lambda i: (0, 0)),
            pl.BlockSpec((D, D), lambda i: (0, 0)),
        ],
        out_specs=pl.BlockSpec((NC, RB, DH), lambda i: (0, i, 0)),
        out_shape=jax.ShapeDtypeStruct((NC, NPAD, DH), jnp.float32),
    )(p, spart, dinv8, bprev, wnext)


def _t3_body(p_ref, s_ref, dinv_ref, b_ref, wg_ref, bg_ref, batch_ref,
             wc1_ref, bc1_ref, wc2_ref, bc2_ref, out_ref, acc):
    pid = pl.program_id(0)
    dinv = dinv_ref[:, :1]
    ps = jnp.concatenate(
        [p_ref[0, :, :] + s_ref[0, :, :], p_ref[1, :, :] + s_ref[1, :, :]],
        axis=1)
    h3 = jnp.maximum(dinv * ps + b_ref[...], 0.0)
    t = jnp.dot(h3, wg_ref[...],
                preferred_element_type=jnp.float32) + bg_ref[...]
    gate = 1.0 / (1.0 + jnp.exp(-t))
    hg = h3 * gate
    hgx = jnp.concatenate([hg, jnp.ones((RB, 8), jnp.float32)], axis=1)
    m = (batch_ref[...] ==
         lax.broadcasted_iota(jnp.int32, (RB, G), 1)).astype(jnp.float32)
    part = lax.dot_general(m, hgx, (((0,), (0,)), ((), ())),
                           preferred_element_type=jnp.float32)

    @pl.when(pid == 0)
    def _():
        acc[...] = jnp.zeros((G, D + 8), jnp.float32)

    acc[...] += part

    @pl.when(pid == GRID - 1)
    def _():
        av = acc[...]
        sums = av[:, :D]
        counts = av[:, D:D + 1]
        hgr = sums / jnp.maximum(counts, 1.0)
        hc = jnp.maximum(
            jnp.dot(hgr, wc1_ref[...], preferred_element_type=jnp.float32)
            + bc1_ref[...], 0.0)
        out_ref[...] = (jnp.dot(hc, wc2_ref[...],
                                preferred_element_type=jnp.float32)
                        + bc2_ref[...])


def _t3(p, spart, dinv8, b3p, wgp, bgp, batchp, wc1p, bc1p, wc2p, bc2p):
    return pl.pallas_call(
        _t3_body,
        grid=(GRID,),
        in_specs=[
            pl.BlockSpec((NC, RB, DH), lambda i: (0, i, 0)),
            pl.BlockSpec((NC, RB, DH), lambda i: (0, i, 0)),
            pl.BlockSpec((RB, 8), lambda i: (i, 0)),
            pl.BlockSpec((1, D), lambda i: (0, 0)),
            pl.BlockSpec((D, D), lambda i: (0, 0)),
            pl.BlockSpec((1, D), lambda i: (0, 0)),
            pl.BlockSpec((RB, 1), lambda i: (i, 0)),
            pl.BlockSpec((D, 128), lambda i: (0, 0)),
            pl.BlockSpec((1, 128), lambda i: (0, 0)),
            pl.BlockSpec((128, 2), lambda i: (0, 0)),
            pl.BlockSpec((1, 2), lambda i: (0, 0)),
        ],
        out_specs=pl.BlockSpec((G, 2), lambda i: (0, 0)),
        out_shape=jax.ShapeDtypeStruct((G, 2), jnp.float32),
        scratch_shapes=[pltpu.VMEM((G, D + 8), jnp.float32)],
    )(p, spart, dinv8, b3p, wgp, bgp, batchp, wc1p, bc1p, wc2p, bc2p)


def _pad_w(w, r, c):
    out = jnp.zeros((r, c), jnp.float32)
    return out.at[:w.shape[0], :w.shape[1]].set(w)


def _pad_b(b, c):
    return jnp.zeros((1, c), jnp.float32).at[0, :b.shape[0]].set(b)


def kernel(x, edge_index, batch, W1, b1, W2, b2, W3, b3, Wg, bg,
           Wc1, bc1, Wc2, bc2):
    src = jnp.full((E_PAD,), N, jnp.int32).at[:E].set(
        edge_index[0].astype(jnp.int32))
    dst = jnp.full((E_PAD,), N, jnp.int32).at[:E].set(
        edge_index[1].astype(jnp.int32))
    xp = jnp.zeros((NPAD, IN), jnp.float32).at[:N].set(x)
    batchp = jnp.full((NPAD, 1), G, jnp.int32).at[:N, 0].set(
        batch.astype(jnp.int32))
    w1p = _pad_w(W1, IN, D)
    w2p, w3p, wgp = (_pad_w(w, D, D) for w in (W2, W3, Wg))
    b1p, b2p, b3p, bgp = (_pad_b(b, D) for b in (b1, b2, b3, bg))
    wc1p = _pad_w(Wc1, D, 128)
    bc1p = _pad_b(bc1, 128)
    wc2p = _pad_w(Wc2, 128, 2)
    bc2p = _pad_b(bc2, 2)

    degp = _degree(dst)
    p1, dinv8 = _t1(degp, xp, w1p)
    s1 = _propagate(src, dst, p1)
    p2 = _t2(p1, s1, dinv8, b1p, w2p)
    s2 = _propagate(src, dst, p2)
    p3 = _t2(p2, s2, dinv8, b2p, w3p)
    s3 = _propagate(src, dst, p3)
    return _t3(p3, s3, dinv8, b3p, wgp, bgp, batchp, wc1p, bc1p, wc2p, bc2p)


# speedup vs baseline: 9.0973x; 1.2909x over previous
import functools

import jax
import jax.numpy as jnp
from jax import lax
from jax.experimental import pallas as pl
from jax.experimental.pallas import tpu as pltpu
from jax.experimental.pallas import tpu_sc as plsc

N = 10000
E = 320000
G = 64
IN = 128
HID = 200
DH = 128
D = 2 * DH
NPAD = 10080
RB = 336
GRID = NPAD // RB

NC, NS = 2, 16
NW = NC * NS
K = 128
E_PAD = ((E + NW * K - 1) // (NW * K)) * (NW * K)
EPS = E_PAD // NS
NCHUNK_P = EPS // K
EPW = E_PAD // NW
NCHUNK_D = EPW // K
WROWS = 632
ZR = 8

_mesh = plsc.VectorSubcoreMesh(core_axis_name="c", subcore_axis_name="s")


@functools.partial(
    pl.kernel,
    out_type=jax.ShapeDtypeStruct((NC, NPAD, 16), jnp.float32),
    mesh=_mesh,
    scratch_types=[
        pltpu.VMEM((K,), jnp.int32),
        pltpu.VMEM((K, 16), jnp.float32),
        pltpu.VMEM((ZR, 16), jnp.float32),
        pltpu.VMEM_SHARED((NPAD, 16), jnp.float32),
        pltpu.SemaphoreType.DMA,
    ],
)
def _degree(dst_hbm, out_hbm, idx_d, ones_b, zbuf, acc, sem):
    c = lax.axis_index("c")
    s = lax.axis_index("s")
    one16 = jnp.ones((16,), jnp.float32)
    zero16 = jnp.zeros((16,), jnp.float32)
    for i in range(K):
        ones_b[i, :] = one16
    for i in range(ZR):
        zbuf[i, :] = zero16
    base_row = jnp.minimum(s * WROWS, NPAD - WROWS)

    def zinit(j, carry):
        pltpu.sync_copy(zbuf, acc.at[pl.ds(base_row + j * ZR, ZR)])
        return carry

    lax.fori_loop(0, WROWS // ZR, zinit, 0)
    plsc.subcore_barrier()

    ebase = (c * NS + s) * EPW

    def body(g, carry):
        pltpu.sync_copy(dst_hbm.at[pl.ds(ebase + g * K, K)], idx_d)
        pltpu.sync_copy(ones_b, acc.at[idx_d], add=True)
        return carry

    lax.fori_loop(0, NCHUNK_D, body, 0)
    plsc.subcore_barrier()
    pltpu.sync_copy(acc.at[pl.ds(base_row, WROWS)],
                    out_hbm.at[c, pl.ds(base_row, WROWS)])


@functools.partial(
    pl.kernel,
    out_type=jax.ShapeDtypeStruct((NC, NPAD, DH), jnp.float32),
    mesh=_mesh,
    scratch_types=[
        pltpu.VMEM((K,), jnp.int32),
        pltpu.VMEM((K,), jnp.int32),
        pltpu.VMEM((K,), jnp.int32),
        pltpu.VMEM((K,), jnp.int32),
        pltpu.VMEM((K, DH), jnp.float32),
        pltpu.VMEM((K, DH), jnp.float32),
        pltpu.VMEM((ZR, DH), jnp.float32),
        pltpu.VMEM_SHARED((NPAD, DH), jnp.float32),
        pltpu.SemaphoreType.DMA,
        pltpu.SemaphoreType.DMA,
    ],
)
def _propagate(src_hbm, dst_hbm, p_hbm, out_hbm, idx_s0, idx_d0, idx_s1,
               idx_d1, rows0, rows1, zbuf, acc, sem0, sem1):
    c = lax.axis_index("c")
    s = lax.axis_index("s")
    zero16 = jnp.zeros((16,), jnp.float32)
    for i in range(ZR):
        for k in range(DH // 16):
            zbuf[i, pl.ds(k * 16, 16)] = zero16
    base_row = jnp.minimum(s * WROWS, NPAD - WROWS)

    def zinit(j, carry):
        pltpu.sync_copy(zbuf, acc.at[pl.ds(base_row + j * ZR, ZR)])
        return carry

    lax.fori_loop(0, WROWS // ZR, zinit, 0)
    plsc.subcore_barrier()

    ebase = s * EPS

    def body(i, carry):
        off = ebase + 2 * i * K
        pltpu.sync_copy(src_hbm.at[pl.ds(off, K)], idx_s0)
        cp0 = pltpu.async_copy(p_hbm.at[c].at[idx_s0], rows0, sem0)
        pltpu.sync_copy(src_hbm.at[pl.ds(off + K, K)], idx_s1)
        cp1 = pltpu.async_copy(p_hbm.at[c].at[idx_s1], rows1, sem1)
        pltpu.sync_copy(dst_hbm.at[pl.ds(off, K)], idx_d0)
        pltpu.sync_copy(dst_hbm.at[pl.ds(off + K, K)], idx_d1)
        cp0.wait()
        pltpu.sync_copy(rows0, acc.at[idx_d0], add=True)
        cp1.wait()
        pltpu.sync_copy(rows1, acc.at[idx_d1], add=True)
        return carry

    lax.fori_loop(0, NCHUNK_P // 2, body, 0)
    plsc.subcore_barrier()
    pltpu.sync_copy(acc.at[pl.ds(base_row, WROWS)],
                    out_hbm.at[c, pl.ds(base_row, WROWS)])


def _t1_body(dp_ref, x_ref, w_ref, p_ref, dinv_ref):
    pid = pl.program_id(0)
    dp0 = dp_ref[0, :, :]
    dp1 = dp_ref[1, :, :]
    deg = 1.0 + dp0[:, :1] + dp1[:, :1]
    row = pid * RB + lax.broadcasted_iota(jnp.int32, (RB, 1), 0)
    dinv = jnp.where(row < N, lax.rsqrt(deg), 0.0)
    z = jnp.dot(x_ref[...], w_ref[...],
                preferred_element_type=jnp.float32) * dinv
    p_ref[0, :, :] = z[:, :DH]
    p_ref[1, :, :] = z[:, DH:]
    dinv_ref[...] = jnp.broadcast_to(dinv, (RB, 8))


def _t1(degp, xp, w1p):
    return pl.pallas_call(
        _t1_body,
        grid=(GRID,),
        in_specs=[
            pl.BlockSpec((NC, RB, 16), lambda i: (0, i, 0)),
            pl.BlockSpec((RB, IN), lambda i: (i, 0)),
            pl.BlockSpec((IN, D), lambda i: (0, 0)),
        ],
        out_specs=[
            pl.BlockSpec((NC, RB, DH), lambda i: (0, i, 0)),
            pl.BlockSpec((RB, 8), lambda i: (i, 0)),
        ],
        out_shape=[
            jax.ShapeDtypeStruct((NC, NPAD, DH), jnp.float32),
            jax.ShapeDtypeStruct((NPAD, 8), jnp.float32),
        ],
    )(degp, xp, w1p)


def _t2_body(p_ref, s_ref, dinv_ref, b_ref, w_ref, out_ref):
    dinv = dinv_ref[:, :1]
    ps = jnp.concatenate(
        [p_ref[0, :, :] + s_ref[0, :, :], p_ref[1, :, :] + s_ref[1, :, :]],
        axis=1)
    h = jnp.maximum(dinv * ps + b_ref[...], 0.0)
    z = jnp.dot(h, w_ref[...], preferred_element_type=jnp.float32) * dinv
    out_ref[0, :, :] = z[:, :DH]
    out_ref[1, :, :] = z[:, DH:]


def _t2(p, spart, dinv8, bprev, wnext):
    return pl.pallas_call(
        _t2_body,
        grid=(GRID,),
        in_specs=[
            pl.BlockSpec((NC, RB, DH), lambda i: (0, i, 0)),
            pl.BlockSpec((NC, RB, DH), lambda i: (0, i, 0)),
            pl.BlockSpec((RB, 8), lambda i: (i, 0)),
            pl.BlockSpec((1, D), lambda i: (0, 0)),
            pl.BlockSpec((D, D), lambda i: (0, 0)),
        ],
        out_specs=pl.BlockSpec((NC, RB, DH), lambda i: (0, i, 0)),
        out_shape=jax.ShapeDtypeStruct((NC, NPAD, DH), jnp.float32),
    )(p, spart, dinv8, bprev, wnext)


def _t3_body(p_ref, s_ref, dinv_ref, b_ref, wg_ref, bg_ref, batch_ref,
             wc1_ref, bc1_ref, wc2_ref, bc2_ref, out_ref, acc):
    pid = pl.program_id(0)
    dinv = dinv_ref[:, :1]
    ps = jnp.concatenate(
        [p_ref[0, :, :] + s_ref[0, :, :], p_ref[1, :, :] + s_ref[1, :, :]],
        axis=1)
    h3 = jnp.maximum(dinv * ps + b_ref[...], 0.0)
    t = jnp.dot(h3, wg_ref[...],
                preferred_element_type=jnp.float32) + bg_ref[...]
    gate = 1.0 / (1.0 + jnp.exp(-t))
    hg = h3 * gate
    hgx = jnp.concatenate([hg, jnp.ones((RB, 8), jnp.float32)], axis=1)
    m = (batch_ref[...] ==
         lax.broadcasted_iota(jnp.int32, (RB, G), 1)).astype(jnp.float32)
    part = lax.dot_general(m, hgx, (((0,), (0,)), ((), ())),
                           preferred_element_type=jnp.float32)

    @pl.when(pid == 0)
    def _():
        acc[...] = jnp.zeros((G, D + 8), jnp.float32)

    acc[...] += part

    @pl.when(pid == GRID - 1)
    def _():
        av = acc[...]
        sums = av[:, :D]
        counts = av[:, D:D + 1]
        hgr = sums / jnp.maximum(counts, 1.0)
        hc = jnp.maximum(
            jnp.dot(hgr, wc1_ref[...], preferred_element_type=jnp.float32)
            + bc1_ref[...], 0.0)
        out_ref[...] = (jnp.dot(hc, wc2_ref[...],
                                preferred_element_type=jnp.float32)
                        + bc2_ref[...])


def _t3(p, spart, dinv8, b3p, wgp, bgp, batchp, wc1p, bc1p, wc2p, bc2p):
    return pl.pallas_call(
        _t3_body,
        grid=(GRID,),
        in_specs=[
            pl.BlockSpec((NC, RB, DH), lambda i: (0, i, 0)),
            pl.BlockSpec((NC, RB, DH), lambda i: (0, i, 0)),
            pl.BlockSpec((RB, 8), lambda i: (i, 0)),
            pl.BlockSpec((1, D), lambda i: (0, 0)),
            pl.BlockSpec((D, D), lambda i: (0, 0)),
            pl.BlockSpec((1, D), lambda i: (0, 0)),
            pl.BlockSpec((RB, 1), lambda i: (i, 0)),
            pl.BlockSpec((D, 128), lambda i: (0, 0)),
            pl.BlockSpec((1, 128), lambda i: (0, 0)),
            pl.BlockSpec((128, 2), lambda i: (0, 0)),
            pl.BlockSpec((1, 2), lambda i: (0, 0)),
        ],
        out_specs=pl.BlockSpec((G, 2), lambda i: (0, 0)),
        out_shape=jax.ShapeDtypeStruct((G, 2), jnp.float32),
        scratch_shapes=[pltpu.VMEM((G, D + 8), jnp.float32)],
    )(p, spart, dinv8, b3p, wgp, bgp, batchp, wc1p, bc1p, wc2p, bc2p)


def _pad_w(w, r, c):
    out = jnp.zeros((r, c), jnp.float32)
    return out.at[:w.shape[0], :w.shape[1]].set(w)


def _pad_b(b, c):
    return jnp.zeros((1, c), jnp.float32).at[0, :b.shape[0]].set(b)


def kernel(x, edge_index, batch, W1, b1, W2, b2, W3, b3, Wg, bg,
           Wc1, bc1, Wc2, bc2):
    src = jnp.full((E_PAD,), N, jnp.int32).at[:E].set(
        edge_index[0].astype(jnp.int32))
    dst = jnp.full((E_PAD,), N, jnp.int32).at[:E].set(
        edge_index[1].astype(jnp.int32))
    xp = jnp.zeros((NPAD, IN), jnp.float32).at[:N].set(x)
    batchp = jnp.full((NPAD, 1), G, jnp.int32).at[:N, 0].set(
        batch.astype(jnp.int32))
    w1p = _pad_w(W1, IN, D)
    w2p, w3p, wgp = (_pad_w(w, D, D) for w in (W2, W3, Wg))
    b1p, b2p, b3p, bgp = (_pad_b(b, D) for b in (b1, b2, b3, bg))
    wc1p = _pad_w(Wc1, D, 128)
    bc1p = _pad_b(bc1, 128)
    wc2p = _pad_w(Wc2, 128, 2)
    bc2p = _pad_b(bc2, 2)

    degp = _degree(dst)
    p1, dinv8 = _t1(degp, xp, w1p)
    s1 = _propagate(src, dst, p1)
    p2 = _t2(p1, s1, dinv8, b1p, w2p)
    s2 = _propagate(src, dst, p2)
    p3 = _t2(p2, s2, dinv8, b2p, w3p)
    s3 = _propagate(src, dst, p3)
    return _t3(p3, s3, dinv8, b3p, wgp, bgp, batchp, wc1p, bc1p, wc2p, bc2p)
